# Initial kernel scaffold; baseline (speedup 1.0000x reference)
import jax
import jax.numpy as jnp
from jax.experimental import pallas as pl


def kernel(x1, features, adj, x2, gene_emb1, gate_w, gate_b, enc_W1, enc_b1, enc_W2, enc_b2, gl_W, g_W1, g_b1, g_W2, g_b2, g_W3, g_b3, d1_W1, d1_b1, d1_g, d1_be, d1_W2, d1_b2, d2_W1, d2_b1, d2_g, d2_be, d2_W2, d2_b2, eps1, eps2):
    raise NotImplementedError("write your pallas kernel here")


# speedup vs baseline: 6.0311x; 6.0311x over previous
import functools

import jax
import jax.numpy as jnp
from jax.experimental import pallas as pl

N_G = 978
PAD = 1024
TOPK = 50
PERS = 8
IB = 64


def _graph_kernel(feat_ref, glW_ref,
                  gW1_ref, gb1_ref, gW2_ref, gb2_ref, gW3_ref, gb3_ref,
                  ge_ref):
    f32 = jnp.float32

    row_i = jax.lax.broadcasted_iota(jnp.int32, (PAD, 1), 0)
    col_i = jax.lax.broadcasted_iota(jnp.int32, (1, PAD), 1)
    row_valid = (row_i < N_G).astype(f32)

    feat = feat_ref[0]

    att = jnp.zeros((PAD, PAD), f32)
    for pi in range(PERS):
        h = jnp.maximum(jnp.dot(feat, glW_ref[pi], preferred_element_type=f32), 0.0)
        att = att + jax.lax.dot_general(
            h, h, (((1,), (1,)), ((), ())), preferred_element_type=f32)
    att = att * (1.0 / PERS)

    bits = jax.lax.bitcast_convert_type(att, jnp.int32)
    key = jnp.where(col_i < N_G, bits, -1)

    def _val_step(_, carry):
        lo, hi = carry
        mid = lo + (hi - lo) // 2
        cnt = jnp.sum((key >= mid).astype(jnp.int32), axis=1, keepdims=True)
        take = cnt >= TOPK
        return jnp.where(take, mid, lo), jnp.where(take, hi, mid)

    lo0 = jnp.zeros((PAD, 1), jnp.int32)
    hi0 = jnp.full((PAD, 1), 0x7F800000, jnp.int32)
    t, _ = jax.lax.fori_loop(0, 31, _val_step, (lo0, hi0))

    gt = key > t
    eq = key == t
    need = TOPK - jnp.sum(gt.astype(jnp.int32), axis=1, keepdims=True)

    def _idx_step(_, carry):
        lo, hi = carry
        mid = lo + (hi - lo) // 2
        cnt = jnp.sum((eq & (col_i <= mid)).astype(jnp.int32), axis=1,
                      keepdims=True)
        take = cnt >= need
        return jnp.where(take, lo, mid), jnp.where(take, mid, hi)

    ilo0 = jnp.full((PAD, 1), -1, jnp.int32)
    ihi0 = jnp.full((PAD, 1), PAD - 1, jnp.int32)
    _, m = jax.lax.fori_loop(0, 10, _idx_step, (ilo0, ihi0))

    sel = gt | (eq & (col_i <= m) & (need > 0))

    neg = jnp.float32(-1e30)
    rowmax = jnp.max(jnp.where(sel, att, neg), axis=1, keepdims=True)
    p = jnp.where(sel, jnp.exp(att - rowmax), 0.0)
    denom = jnp.sum(p, axis=1, keepdims=True)
    A = (p / jnp.maximum(denom, 1e-30)) * row_valid

    eye = ((row_i == col_i) & (row_i < N_G)).astype(f32)
    A1 = A + eye
    A1T = A1.T
    deg_c = jnp.sum(A1T, axis=1, keepdims=True)
    deg_l = jnp.sum(A1, axis=0, keepdims=True)
    dis_c = jax.lax.rsqrt(jnp.maximum(deg_c, 1e-12))
    dis_l = jax.lax.rsqrt(jnp.maximum(deg_l, 1e-12))
    AhatT = A1T * dis_c * dis_l

    X = feat
    h1 = jnp.dot(AhatT, jnp.dot(X, gW1_ref[:], preferred_element_type=f32),
                 preferred_element_type=f32) + gb1_ref[:]
    h1 = jnp.maximum(h1, 0.0)
    h2 = jnp.dot(AhatT, jnp.dot(h1, gW2_ref[:], preferred_element_type=f32),
                 preferred_element_type=f32) + gb2_ref[:]
    h2 = jnp.maximum(h2, 0.0)
    h3 = jnp.dot(AhatT, jnp.dot(h2, gW3_ref[:], preferred_element_type=f32),
                 preferred_element_type=f32) + gb3_ref[:]

    ge_ref[0] = jnp.sum(h3 * row_valid, axis=0, keepdims=True) * (1.0 / N_G)


def _decode_kernel(ge_ref, eps_ref, W1_ref, b1_ref, gam_ref, be_ref,
                   W2_ref, b2_ref, rec_ref, mu_ref, std_ref):
    f32 = jnp.float32
    ge = ge_ref[0]
    mu = ge[:, :IB]
    raw = ge[:, IB:] - float(IB)
    std = jnp.maximum(raw, 0.0) + jnp.log1p(jnp.exp(-jnp.abs(raw)))
    z = mu + eps_ref[0] * std
    h = jnp.dot(z, W1_ref[0], preferred_element_type=f32) + b1_ref[0]
    m = jnp.mean(h, axis=0, keepdims=True)
    v = jnp.mean((h - m) ** 2, axis=0, keepdims=True)
    h = gam_ref[0] * (h - m) / jnp.sqrt(v + 1e-5) + be_ref[0]
    h = jnp.maximum(h, 0.0)
    out = jnp.dot(h, W2_ref[0], preferred_element_type=f32) + b2_ref[0]
    rec_ref[0] = jnp.maximum(out, 0.0)
    mu_ref[0] = mu
    std_ref[0] = std


@functools.partial(jax.jit, static_argnames=("interpret",))
def _run(x1, x2, gene_emb1, gate_w, gate_b, enc_W1, enc_b1, enc_W2, enc_b2,
         gl_W, g_W1, g_b1, g_W2, g_b2, g_W3, g_b3,
         d1_W1, d1_b1, d1_g, d1_be, d1_W2, d1_b2,
         d2_W1, d2_b1, d2_g, d2_be, d2_W2, d2_b2, eps1, eps2,
         interpret=False):
    f32 = jnp.float32
    B = x1.shape[0]

    def encode(x):
        gate = jax.nn.sigmoid(x[..., None] * gate_w + gate_b)
        H = gate * gene_emb1[None]
        H = jax.nn.gelu(H @ enc_W1 + enc_b1, approximate=False)
        return H @ enc_W2 + enc_b2

    feat = jnp.concatenate([encode(x1), encode(x2)], axis=0)
    feat = jnp.pad(feat, ((0, 0), (0, PAD - N_G), (0, 0)))

    full = lambda s: pl.BlockSpec(s, lambda i: (0,) * len(s))

    def r1(v):
        return v.reshape(1, -1)

    in_specs = [
        pl.BlockSpec((1, PAD, feat.shape[2]), lambda i: (i, 0, 0)),
        full(gl_W.shape),
        full(g_W1.shape), full((1, g_b1.shape[0])),
        full(g_W2.shape), full((1, g_b2.shape[0])),
        full(g_W3.shape), full((1, g_b3.shape[0])),
    ]
    ge = pl.pallas_call(
        _graph_kernel,
        grid=(2 * B,),
        in_specs=in_specs,
        out_specs=pl.BlockSpec((1, 1, 2 * IB), lambda i: (i, 0, 0)),
        out_shape=jax.ShapeDtypeStruct((2 * B, 1, 2 * IB), f32),
        interpret=interpret,
    )(feat, gl_W, g_W1, r1(g_b1), g_W2, r1(g_b2), g_W3, r1(g_b3))

    ge2 = ge.reshape(2, B, 2 * IB)
    epss = jnp.stack([eps1, eps2])
    W1s = jnp.stack([d1_W1, d2_W1])
    b1s = jnp.stack([d1_b1, d2_b1]).reshape(2, 1, -1)
    gs = jnp.stack([d1_g, d2_g]).reshape(2, 1, -1)
    bes = jnp.stack([d1_be, d2_be]).reshape(2, 1, -1)
    W2s = jnp.stack([d1_W2, d2_W2])
    b2s = jnp.stack([d1_b2, d2_b2]).reshape(2, 1, -1)

    blk = lambda s: pl.BlockSpec((1,) + s[1:], lambda i: (i,) + (0,) * (len(s) - 1))
    rec, mu, std = pl.pallas_call(
        _decode_kernel,
        grid=(2,),
        in_specs=[blk(ge2.shape), blk(epss.shape), blk(W1s.shape),
                  blk(b1s.shape), blk(gs.shape), blk(bes.shape),
                  blk(W2s.shape), blk(b2s.shape)],
        out_specs=[blk((2, B, N_G)), blk((2, B, IB)), blk((2, B, IB))],
        out_shape=[jax.ShapeDtypeStruct((2, B, N_G), f32),
                   jax.ShapeDtypeStruct((2, B, IB), f32),
                   jax.ShapeDtypeStruct((2, B, IB), f32)],
        interpret=interpret,
    )(ge2, epss, W1s, b1s, gs, bes, W2s, b2s)

    return (rec[0], rec[1], mu[0], std[0], mu[1], std[1])


def kernel(x1, features, adj, x2, gene_emb1, gate_w, gate_b, enc_W1, enc_b1,
           enc_W2, enc_b2, gl_W, g_W1, g_b1, g_W2, g_b2, g_W3, g_b3,
           d1_W1, d1_b1, d1_g, d1_be, d1_W2, d1_b2,
           d2_W1, d2_b1, d2_g, d2_be, d2_W2, d2_b2, eps1, eps2):
    del features, adj
    return _run(x1, x2, gene_emb1, gate_w, gate_b, enc_W1, enc_b1, enc_W2,
                enc_b2, gl_W, g_W1, g_b1, g_W2, g_b2, g_W3, g_b3,
                d1_W1, d1_b1, d1_g, d1_be, d1_W2, d1_b2,
                d2_W1, d2_b1, d2_g, d2_be, d2_W2, d2_b2, eps1, eps2)


# speedup vs baseline: 6.1841x; 1.0254x over previous
import functools

import jax
import jax.numpy as jnp
from jax.experimental import pallas as pl

N_G = 978
PAD = 1024
TOPK = 50
PERS = 8
IB = 64


def _graph_kernel(feat_ref, glW_ref,
                  gW1_ref, gb1_ref, gW2_ref, gb2_ref, gW3_ref, gb3_ref,
                  ge_ref):
    f32 = jnp.float32

    row_i = jax.lax.broadcasted_iota(jnp.int32, (PAD, 1), 0)
    col_i = jax.lax.broadcasted_iota(jnp.int32, (1, PAD), 1)
    row_valid = (row_i < N_G).astype(f32)

    feat = feat_ref[0]

    att = jnp.zeros((PAD, PAD), f32)
    for pi in range(PERS):
        h = jnp.maximum(jnp.dot(feat, glW_ref[pi], preferred_element_type=f32), 0.0)
        att = att + jax.lax.dot_general(
            h, h, (((1,), (1,)), ((), ())), preferred_element_type=f32)
    att = att * (1.0 / PERS)

    bits = jax.lax.bitcast_convert_type(att, jnp.int32)
    key_hi = jnp.where(col_i < N_G, bits >> 16, -1).astype(jnp.int16)

    def _hi_step(_, carry):
        lo, hi = carry
        mid = lo + ((hi - lo) >> 1)
        cnt = jnp.sum((key_hi >= mid.astype(jnp.int16)).astype(jnp.int16),
                      axis=1, keepdims=True).astype(jnp.int32)
        take = cnt >= TOPK
        return jnp.where(take, mid, lo), jnp.where(take, hi, mid)

    lo0 = jnp.zeros((PAD, 1), jnp.int32)
    hi0 = jnp.full((PAD, 1), 0x7F81, jnp.int32)
    t_hi, _ = jax.lax.fori_loop(0, 15, _hi_step, (lo0, hi0))
    t_hi16 = t_hi.astype(jnp.int16)

    hi_gt = key_hi > t_hi16
    hi_eq = key_hi == t_hi16
    n_hi_gt = jnp.sum(hi_gt.astype(jnp.int16), axis=1,
                      keepdims=True).astype(jnp.int32)
    need_lo = TOPK - n_hi_gt

    low = jnp.bitwise_and(bits, 0xFFFF) - 0x8000
    key_lo = jnp.where(hi_eq, low.astype(jnp.int16), jnp.int16(-0x8000))

    def _lo_step(_, carry):
        lo, hi = carry
        mid = lo + ((hi - lo) >> 1)
        cnt = jnp.sum((key_lo >= mid.astype(jnp.int16)).astype(jnp.int16),
                      axis=1, keepdims=True).astype(jnp.int32)
        take = cnt >= need_lo
        return jnp.where(take, mid, lo), jnp.where(take, hi, mid)

    llo0 = jnp.full((PAD, 1), -0x8000, jnp.int32)
    lhi0 = jnp.full((PAD, 1), 0x8000, jnp.int32)
    t_lo, _ = jax.lax.fori_loop(0, 16, _lo_step, (llo0, lhi0))
    t_lo16 = t_lo.astype(jnp.int16)

    gt = hi_gt | (hi_eq & (key_lo > t_lo16))
    eq = hi_eq & (key_lo == t_lo16)
    need = TOPK - jnp.sum(gt.astype(jnp.int16), axis=1,
                          keepdims=True).astype(jnp.int32)

    tri = (row_i <= col_i).astype(jnp.bfloat16)
    csum = jnp.dot(eq.astype(jnp.bfloat16), tri, preferred_element_type=f32)
    sel = gt | (eq & (csum <= need.astype(f32)))

    neg = jnp.float32(-1e30)
    rowmax = jnp.max(jnp.where(sel, att, neg), axis=1, keepdims=True)
    p = jnp.where(sel, jnp.exp(att - rowmax), 0.0)
    denom = jnp.sum(p, axis=1, keepdims=True)
    A = (p / jnp.maximum(denom, 1e-30)) * row_valid

    eye = ((row_i == col_i) & (row_i < N_G)).astype(f32)
    A1 = A + eye
    A1T = A1.T
    deg_c = jnp.sum(A1T, axis=1, keepdims=True)
    deg_l = jnp.sum(A1, axis=0, keepdims=True)
    dis_c = jax.lax.rsqrt(jnp.maximum(deg_c, 1e-12))
    dis_l = jax.lax.rsqrt(jnp.maximum(deg_l, 1e-12))
    AhatT = A1T * dis_c * dis_l

    X = feat
    h1 = jnp.dot(AhatT, jnp.dot(X, gW1_ref[:], preferred_element_type=f32),
                 preferred_element_type=f32) + gb1_ref[:]
    h1 = jnp.maximum(h1, 0.0)
    h2 = jnp.dot(AhatT, jnp.dot(h1, gW2_ref[:], preferred_element_type=f32),
                 preferred_element_type=f32) + gb2_ref[:]
    h2 = jnp.maximum(h2, 0.0)
    h3 = jnp.dot(AhatT, jnp.dot(h2, gW3_ref[:], preferred_element_type=f32),
                 preferred_element_type=f32) + gb3_ref[:]

    ge_ref[0] = jnp.sum(h3 * row_valid, axis=0, keepdims=True) * (1.0 / N_G)


def _decode_kernel(ge_ref, eps_ref, W1_ref, b1_ref, gam_ref, be_ref,
                   W2_ref, b2_ref, rec_ref, mu_ref, std_ref):
    f32 = jnp.float32
    ge = ge_ref[0]
    mu = ge[:, :IB]
    raw = ge[:, IB:] - float(IB)
    std = jnp.maximum(raw, 0.0) + jnp.log1p(jnp.exp(-jnp.abs(raw)))
    z = mu + eps_ref[0] * std
    h = jnp.dot(z, W1_ref[0], preferred_element_type=f32) + b1_ref[0]
    m = jnp.mean(h, axis=0, keepdims=True)
    v = jnp.mean((h - m) ** 2, axis=0, keepdims=True)
    h = gam_ref[0] * (h - m) / jnp.sqrt(v + 1e-5) + be_ref[0]
    h = jnp.maximum(h, 0.0)
    out = jnp.dot(h, W2_ref[0], preferred_element_type=f32) + b2_ref[0]
    rec_ref[0] = jnp.maximum(out, 0.0)
    mu_ref[0] = mu
    std_ref[0] = std


@functools.partial(jax.jit, static_argnames=("interpret",))
def _run(x1, x2, gene_emb1, gate_w, gate_b, enc_W1, enc_b1, enc_W2, enc_b2,
         gl_W, g_W1, g_b1, g_W2, g_b2, g_W3, g_b3,
         d1_W1, d1_b1, d1_g, d1_be, d1_W2, d1_b2,
         d2_W1, d2_b1, d2_g, d2_be, d2_W2, d2_b2, eps1, eps2,
         interpret=False):
    f32 = jnp.float32
    B = x1.shape[0]

    def encode(x):
        gate = jax.nn.sigmoid(x[..., None] * gate_w + gate_b)
        H = gate * gene_emb1[None]
        H = jax.nn.gelu(H @ enc_W1 + enc_b1, approximate=False)
        return H @ enc_W2 + enc_b2

    feat = jnp.concatenate([encode(x1), encode(x2)], axis=0)
    feat = jnp.pad(feat, ((0, 0), (0, PAD - N_G), (0, 0)))

    full = lambda s: pl.BlockSpec(s, lambda i: (0,) * len(s))

    def r1(v):
        return v.reshape(1, -1)

    in_specs = [
        pl.BlockSpec((1, PAD, feat.shape[2]), lambda i: (i, 0, 0)),
        full(gl_W.shape),
        full(g_W1.shape), full((1, g_b1.shape[0])),
        full(g_W2.shape), full((1, g_b2.shape[0])),
        full(g_W3.shape), full((1, g_b3.shape[0])),
    ]
    ge = pl.pallas_call(
        _graph_kernel,
        grid=(2 * B,),
        in_specs=in_specs,
        out_specs=pl.BlockSpec((1, 1, 2 * IB), lambda i: (i, 0, 0)),
        out_shape=jax.ShapeDtypeStruct((2 * B, 1, 2 * IB), f32),
        interpret=interpret,
    )(feat, gl_W, g_W1, r1(g_b1), g_W2, r1(g_b2), g_W3, r1(g_b3))

    ge2 = ge.reshape(2, B, 2 * IB)
    epss = jnp.stack([eps1, eps2])
    W1s = jnp.stack([d1_W1, d2_W1])
    b1s = jnp.stack([d1_b1, d2_b1]).reshape(2, 1, -1)
    gs = jnp.stack([d1_g, d2_g]).reshape(2, 1, -1)
    bes = jnp.stack([d1_be, d2_be]).reshape(2, 1, -1)
    W2s = jnp.stack([d1_W2, d2_W2])
    b2s = jnp.stack([d1_b2, d2_b2]).reshape(2, 1, -1)

    blk = lambda s: pl.BlockSpec((1,) + s[1:], lambda i: (i,) + (0,) * (len(s) - 1))
    rec, mu, std = pl.pallas_call(
        _decode_kernel,
        grid=(2,),
        in_specs=[blk(ge2.shape), blk(epss.shape), blk(W1s.shape),
                  blk(b1s.shape), blk(gs.shape), blk(bes.shape),
                  blk(W2s.shape), blk(b2s.shape)],
        out_specs=[blk((2, B, N_G)), blk((2, B, IB)), blk((2, B, IB))],
        out_shape=[jax.ShapeDtypeStruct((2, B, N_G), f32),
                   jax.ShapeDtypeStruct((2, B, IB), f32),
                   jax.ShapeDtypeStruct((2, B, IB), f32)],
        interpret=interpret,
    )(ge2, epss, W1s, b1s, gs, bes, W2s, b2s)

    return (rec[0], rec[1], mu[0], std[0], mu[1], std[1])


def kernel(x1, features, adj, x2, gene_emb1, gate_w, gate_b, enc_W1, enc_b1,
           enc_W2, enc_b2, gl_W, g_W1, g_b1, g_W2, g_b2, g_W3, g_b3,
           d1_W1, d1_b1, d1_g, d1_be, d1_W2, d1_b2,
           d2_W1, d2_b1, d2_g, d2_be, d2_W2, d2_b2, eps1, eps2):
    del features, adj
    return _run(x1, x2, gene_emb1, gate_w, gate_b, enc_W1, enc_b1, enc_W2,
                enc_b2, gl_W, g_W1, g_b1, g_W2, g_b2, g_W3, g_b3,
                d1_W1, d1_b1, d1_g, d1_be, d1_W2, d1_b2,
                d2_W1, d2_b1, d2_g, d2_be, d2_W2, d2_b2, eps1, eps2)


# speedup vs baseline: 8.1622x; 1.3199x over previous
import functools

import jax
import jax.numpy as jnp
from jax.experimental import pallas as pl

N_G = 978
PAD = 1024
TOPK = 50
PERS = 8
IB = 64


def _graph_kernel(feat_ref, glW_ref,
                  gW1_ref, gb1_ref, gW2_ref, gb2_ref, gW3_ref, gb3_ref,
                  ge_ref):
    f32 = jnp.float32

    row_i = jax.lax.broadcasted_iota(jnp.int32, (PAD, 1), 0)
    col_i = jax.lax.broadcasted_iota(jnp.int32, (1, PAD), 1)
    row_valid = (row_i < N_G).astype(f32)

    feat = feat_ref[0]

    att = jnp.zeros((PAD, PAD), f32)
    for pi in range(PERS):
        h = jnp.maximum(jnp.dot(feat, glW_ref[pi], preferred_element_type=f32), 0.0)
        att = att + jax.lax.dot_general(
            h, h, (((1,), (1,)), ((), ())), preferred_element_type=f32)
    att = att * (1.0 / PERS)

    bits = jax.lax.bitcast_convert_type(att, jnp.int32)
    key = jnp.where(col_i < N_G, bits, -1)
    ones8 = jnp.ones((PAD, 8), jnp.bfloat16)
    topk_f = jnp.float32(TOPK)

    def _count(mask_bf):
        return jnp.dot(mask_bf, ones8, preferred_element_type=f32)[:, 0:1]

    def _val_step(_, carry):
        lo, hi = carry
        mid = lo + ((hi - lo) >> 1)
        cnt = _count((key >= mid).astype(jnp.bfloat16))
        take = cnt >= topk_f
        return jnp.where(take, mid, lo), jnp.where(take, hi, mid)

    lo0 = jnp.zeros((PAD, 1), jnp.int32)
    hi0 = jnp.full((PAD, 1), 0x7F800000, jnp.int32)
    t, _ = jax.lax.fori_loop(0, 31, _val_step, (lo0, hi0))

    gt = key > t
    eq = key == t
    need = topk_f - _count(gt.astype(jnp.bfloat16))

    tri = (row_i <= col_i).astype(jnp.bfloat16)
    csum = jnp.dot(eq.astype(jnp.bfloat16), tri, preferred_element_type=f32)
    sel = gt | (eq & (csum <= need))

    neg = jnp.float32(-1e30)
    rowmax = jnp.max(jnp.where(sel, att, neg), axis=1, keepdims=True)
    p = jnp.where(sel, jnp.exp(att - rowmax), 0.0)
    denom = jnp.sum(p, axis=1, keepdims=True)
    A = (p / jnp.maximum(denom, 1e-30)) * row_valid

    eye = ((row_i == col_i) & (row_i < N_G)).astype(f32)
    A1 = A + eye
    A1T = A1.T
    deg_l = jnp.sum(A1, axis=0, keepdims=True)
    dis_l = jax.lax.rsqrt(jnp.maximum(deg_l, 1e-12))
    dis_c = dis_l.reshape(PAD, 1)
    AhatT = (A1T * dis_l) * dis_c

    X = feat
    h1 = jnp.dot(AhatT, jnp.dot(X, gW1_ref[:], preferred_element_type=f32),
                 preferred_element_type=f32) + gb1_ref[:]
    h1 = jnp.maximum(h1, 0.0)
    h2 = jnp.dot(AhatT, jnp.dot(h1, gW2_ref[:], preferred_element_type=f32),
                 preferred_element_type=f32) + gb2_ref[:]
    h2 = jnp.maximum(h2, 0.0)
    h3 = jnp.dot(AhatT, jnp.dot(h2, gW3_ref[:], preferred_element_type=f32),
                 preferred_element_type=f32) + gb3_ref[:]

    ge_ref[0] = jnp.sum(h3 * row_valid, axis=0, keepdims=True) / jnp.float32(N_G)


def _decode_kernel(ge_ref, eps_ref, W1_ref, b1_ref, gam_ref, be_ref,
                   W2_ref, b2_ref, rec_ref, mu_ref, std_ref):
    f32 = jnp.float32
    ge = ge_ref[0]
    mu = ge[:, :IB]
    raw = ge[:, IB:] - float(IB)
    std = jnp.maximum(raw, 0.0) + jnp.log1p(jnp.exp(-jnp.abs(raw)))
    z = mu + eps_ref[0] * std
    h = jnp.dot(z, W1_ref[0], preferred_element_type=f32) + b1_ref[0]
    m = jnp.mean(h, axis=0, keepdims=True)
    v = jnp.mean((h - m) ** 2, axis=0, keepdims=True)
    h = gam_ref[0] * (h - m) / jnp.sqrt(v + 1e-5) + be_ref[0]
    h = jnp.maximum(h, 0.0)
    out = jnp.dot(h, W2_ref[0], preferred_element_type=f32) + b2_ref[0]
    rec_ref[0] = jnp.maximum(out, 0.0)
    mu_ref[0] = mu
    std_ref[0] = std


@functools.partial(jax.jit, static_argnames=("interpret",))
def _run(x1, x2, gene_emb1, gate_w, gate_b, enc_W1, enc_b1, enc_W2, enc_b2,
         gl_W, g_W1, g_b1, g_W2, g_b2, g_W3, g_b3,
         d1_W1, d1_b1, d1_g, d1_be, d1_W2, d1_b2,
         d2_W1, d2_b1, d2_g, d2_be, d2_W2, d2_b2, eps1, eps2,
         interpret=False):
    f32 = jnp.float32
    B = x1.shape[0]

    def encode(x):
        gate = jax.nn.sigmoid(x[..., None] * gate_w + gate_b)
        H = gate * gene_emb1[None]
        H = jax.nn.gelu(H @ enc_W1 + enc_b1, approximate=False)
        return H @ enc_W2 + enc_b2

    feat = jnp.concatenate([encode(x1), encode(x2)], axis=0)
    feat = jnp.pad(feat, ((0, 0), (0, PAD - N_G), (0, 0)))

    full = lambda s: pl.BlockSpec(s, lambda i: (0,) * len(s))

    def r1(v):
        return v.reshape(1, -1)

    in_specs = [
        pl.BlockSpec((1, PAD, feat.shape[2]), lambda i: (i, 0, 0)),
        full(gl_W.shape),
        full(g_W1.shape), full((1, g_b1.shape[0])),
        full(g_W2.shape), full((1, g_b2.shape[0])),
        full(g_W3.shape), full((1, g_b3.shape[0])),
    ]
    ge = pl.pallas_call(
        _graph_kernel,
        grid=(2 * B,),
        in_specs=in_specs,
        out_specs=pl.BlockSpec((1, 1, 2 * IB), lambda i: (i, 0, 0)),
        out_shape=jax.ShapeDtypeStruct((2 * B, 1, 2 * IB), f32),
        interpret=interpret,
    )(feat, gl_W, g_W1, r1(g_b1), g_W2, r1(g_b2), g_W3, r1(g_b3))

    ge2 = ge.reshape(2, B, 2 * IB)
    epss = jnp.stack([eps1, eps2])
    W1s = jnp.stack([d1_W1, d2_W1])
    b1s = jnp.stack([d1_b1, d2_b1]).reshape(2, 1, -1)
    gs = jnp.stack([d1_g, d2_g]).reshape(2, 1, -1)
    bes = jnp.stack([d1_be, d2_be]).reshape(2, 1, -1)
    W2s = jnp.stack([d1_W2, d2_W2])
    b2s = jnp.stack([d1_b2, d2_b2]).reshape(2, 1, -1)

    blk = lambda s: pl.BlockSpec((1,) + s[1:], lambda i: (i,) + (0,) * (len(s) - 1))
    rec, mu, std = pl.pallas_call(
        _decode_kernel,
        grid=(2,),
        in_specs=[blk(ge2.shape), blk(epss.shape), blk(W1s.shape),
                  blk(b1s.shape), blk(gs.shape), blk(bes.shape),
                  blk(W2s.shape), blk(b2s.shape)],
        out_specs=[blk((2, B, N_G)), blk((2, B, IB)), blk((2, B, IB))],
        out_shape=[jax.ShapeDtypeStruct((2, B, N_G), f32),
                   jax.ShapeDtypeStruct((2, B, IB), f32),
                   jax.ShapeDtypeStruct((2, B, IB), f32)],
        interpret=interpret,
    )(ge2, epss, W1s, b1s, gs, bes, W2s, b2s)

    return (rec[0], rec[1], mu[0], std[0], mu[1], std[1])


def kernel(x1, features, adj, x2, gene_emb1, gate_w, gate_b, enc_W1, enc_b1,
           enc_W2, enc_b2, gl_W, g_W1, g_b1, g_W2, g_b2, g_W3, g_b3,
           d1_W1, d1_b1, d1_g, d1_be, d1_W2, d1_b2,
           d2_W1, d2_b1, d2_g, d2_be, d2_W2, d2_b2, eps1, eps2):
    del features, adj
    return _run(x1, x2, gene_emb1, gate_w, gate_b, enc_W1, enc_b1, enc_W2,
                enc_b2, gl_W, g_W1, g_b1, g_W2, g_b2, g_W3, g_b3,
                d1_W1, d1_b1, d1_g, d1_be, d1_W2, d1_b2,
                d2_W1, d2_b1, d2_g, d2_be, d2_W2, d2_b2, eps1, eps2)
